# baseline (speedup 1.0000x reference)
import dataclasses
import functools
import jax
import jax.numpy as jnp
from jax import lax
from jax.experimental import pallas as pl
from jax.experimental.pallas import tpu as pltpu
from jax.experimental.pallas import tpu_sc as plsc

N_ROWS = 32768
D = 128
B = 16
L = 16
NVEC = D // L
NWC = 16
SEGC = B // 2
NR = 256
PAY = NR - 8
NBUF = 3
MAXC = 5
UNROLL = 4


def _sc_segment_means(x, batch_lengths):
    mesh = plsc.VectorSubcoreMesh(
        core_axis_name="c", subcore_axis_name="s", num_cores=2, num_subcores=16
    )
    cp = pltpu.CompilerParams()
    if "needs_layout_passes" in pltpu.CompilerParams.__dataclass_fields__:
        cp = dataclasses.replace(cp, needs_layout_passes=False)

    @functools.partial(
        pl.kernel,
        out_type=jax.ShapeDtypeStruct((B, D), jnp.float32),
        mesh=mesh,
        scratch_types=[
            pltpu.VMEM((B,), jnp.int32),
            pltpu.VMEM((B,), jnp.int32),
            pltpu.VMEM((NR, D), jnp.float32),
            pltpu.VMEM((NR, D), jnp.float32),
            pltpu.VMEM((NR, D), jnp.float32),
            pltpu.VMEM((B, D), jnp.float32),
            pltpu.VMEM((D,), jnp.float32),
            pltpu.VMEM_SHARED((B, D), jnp.float32),
            pltpu.SemaphoreType.DMA,
            pltpu.SemaphoreType.DMA,
            pltpu.SemaphoreType.DMA,
        ],
        compiler_params=cp,
    )
    def kern(x_hbm, len_hbm, out_hbm, len_vmem, idx_vmem,
             buf0, buf1, buf2, part, row_vmem, shared, sem0, sem1, sem2):
        c = lax.axis_index("c")
        s = lax.axis_index("s")

        pltpu.sync_copy(len_hbm, len_vmem)
        lv = len_vmem[...]
        ends = plsc.cumsum(lv)
        lanes = lax.iota(jnp.int32, L)
        idx_vmem[...] = lanes
        zeros_i = jnp.zeros((L,), jnp.int32)

        def lane(vec, i):
            return jnp.sum(jnp.where(lanes == i, vec, zeros_i))

        seg0 = c * SEGC
        core_lo = lane(ends, seg0 - 1)
        core_hi = lane(ends, seg0 + SEGC - 1)

        core_rows = core_hi - core_lo
        rows_per_w = (core_rows + NWC - 1) // NWC
        r0 = core_lo + s * rows_per_w
        r1 = jnp.minimum(r0 + rows_per_w, core_hi)

        bufs = [buf0, buf1, buf2, buf0, buf1]
        sems = [sem0, sem1, sem2, sem0, sem1]
        gstarts, aligneds, glens = [], [], []
        for k in range(MAXC):
            gstart = r0 + k * PAY
            aligned = jnp.minimum((gstart // 8) * 8, N_ROWS - NR)
            glen = jnp.minimum(PAY, r1 - gstart)
            gstarts.append(gstart)
            aligneds.append(aligned)
            glens.append(glen)

        def start_copy(k):
            pltpu.async_copy(
                x_hbm.at[pl.ds(aligneds[k], NR)], bufs[k], sems[k]
            )

        def wait_copy(k):
            pltpu.make_async_copy(
                x_hbm.at[pl.ds(aligneds[k], NR)], bufs[k], sems[k]
            ).wait()

        for k in range(NBUF):
            @pl.when(glens[k] > 0)
            def _(k=k):
                start_copy(k)

        seg_end = [lane(ends, seg0 + i) for i in range(SEGC)]
        seg_len = [lane(lv, seg0 + i) for i in range(SEGC)]

        zf = jnp.zeros((L,), jnp.float32)
        for i in range(B):
            for j in range(NVEC):
                part[i, pl.ds(L * j, L)] = zf

        @pl.when(s == 0)
        def _():
            pltpu.sync_copy(part, shared)
        plsc.subcore_barrier()

        def process(k):
            buf = bufs[k]
            gstart, glen = gstarts[k], glens[k]
            off = gstart - aligneds[k]
            gend = gstart + glen
            for i in range(SEGC):
                r = seg0 + i
                lo = jnp.maximum(seg_end[i] - seg_len[i], gstart)
                hi = jnp.minimum(seg_end[i], gend)
                n = hi - lo

                @pl.when(n > 0)
                def _():
                    base = off + (lo - gstart)
                    nu = n // UNROLL

                    def quad_body(q, a):
                        p = base + q * UNROLL
                        for u in range(UNROLL):
                            a = tuple(
                                a[j] + buf[p + u, pl.ds(L * j, L)]
                                for j in range(NVEC)
                            )
                        return a

                    def row_body(q, a):
                        p = base + q
                        return tuple(
                            a[j] + buf[p, pl.ds(L * j, L)]
                            for j in range(NVEC)
                        )

                    accs = lax.fori_loop(
                        0, nu, quad_body,
                        tuple(zf for _ in range(NVEC)),
                    )
                    accs = lax.fori_loop(nu * UNROLL, n, row_body, accs)
                    for j in range(NVEC):
                        part[r, pl.ds(L * j, L)] = (
                            part[r, pl.ds(L * j, L)] + accs[j]
                        )

        for k in range(MAXC):
            @pl.when(glens[k] > 0)
            def _(k=k):
                wait_copy(k)
                process(k)
            if k + NBUF < MAXC:
                @pl.when(glens[k + NBUF] > 0)
                def _(k=k):
                    start_copy(k + NBUF)

        pltpu.sync_copy(part, shared.at[idx_vmem], add=True)
        plsc.subcore_barrier()

        @pl.when(s < SEGC)
        def _():
            r = seg0 + s
            pltpu.sync_copy(shared.at[r], row_vmem)
            den = jnp.full((L,), lane(lv, r), jnp.float32)
            for j in range(NVEC):
                row_vmem[pl.ds(L * j, L)] = row_vmem[pl.ds(L * j, L)] / den
            pltpu.sync_copy(row_vmem, out_hbm.at[r])

    return kern(x, batch_lengths)


def kernel(x, batch_lengths):
    lens = batch_lengths.astype(jnp.int32)
    return _sc_segment_means(x, lens)


# speedup vs baseline: 1.2166x; 1.2166x over previous
import dataclasses
import functools
import jax
import jax.numpy as jnp
from jax import lax
from jax.experimental import pallas as pl
from jax.experimental.pallas import tpu as pltpu
from jax.experimental.pallas import tpu_sc as plsc

N_ROWS = 32768
D = 128
B = 16
L = 16
NVEC = D // L
NWC = 16
SEGC = B // 2
NR = 256
PAY = NR - 8
NBUF = 3
MAXC = 5
UNROLL = 4


def _sc_segment_means(x, batch_lengths):
    mesh = plsc.VectorSubcoreMesh(
        core_axis_name="c", subcore_axis_name="s", num_cores=2, num_subcores=16
    )
    cp = pltpu.CompilerParams()
    if "needs_layout_passes" in pltpu.CompilerParams.__dataclass_fields__:
        cp = dataclasses.replace(cp, needs_layout_passes=False)

    @functools.partial(
        pl.kernel,
        out_type=jax.ShapeDtypeStruct((B, D), jnp.float32),
        mesh=mesh,
        scratch_types=[
            pltpu.VMEM((B,), jnp.int32),
            pltpu.VMEM((B,), jnp.int32),
            pltpu.VMEM((NR, D), jnp.float32),
            pltpu.VMEM((NR, D), jnp.float32),
            pltpu.VMEM((NR, D), jnp.float32),
            pltpu.VMEM((B, D), jnp.float32),
            pltpu.VMEM((D,), jnp.float32),
            pltpu.VMEM_SHARED((B, D), jnp.float32),
            pltpu.SemaphoreType.DMA,
            pltpu.SemaphoreType.DMA,
            pltpu.SemaphoreType.DMA,
        ],
        compiler_params=cp,
    )
    def kern(x_hbm, len_hbm, out_hbm, len_vmem, idx_vmem,
             buf0, buf1, buf2, part, row_vmem, shared, sem0, sem1, sem2):
        c = lax.axis_index("c")
        s = lax.axis_index("s")

        pltpu.sync_copy(len_hbm, len_vmem)
        lv = len_vmem[...]
        ends = plsc.cumsum(lv)
        lanes = lax.iota(jnp.int32, L)
        idx_vmem[...] = lanes
        zeros_i = jnp.zeros((L,), jnp.int32)

        def lane(vec, i):
            return jnp.sum(jnp.where(lanes == i, vec, zeros_i))

        seg0 = c * SEGC
        core_lo = lane(ends, seg0 - 1)
        core_hi = lane(ends, seg0 + SEGC - 1)

        core_rows = core_hi - core_lo
        rows_per_w = (core_rows + NWC - 1) // NWC
        r0 = core_lo + s * rows_per_w
        r1 = jnp.minimum(r0 + rows_per_w, core_hi)

        bufs = [buf0, buf1, buf2, buf0, buf1]
        sems = [sem0, sem1, sem2, sem0, sem1]
        gstarts, aligneds, glens = [], [], []
        for k in range(MAXC):
            gstart = r0 + k * PAY
            aligned = jnp.minimum((gstart // 8) * 8, N_ROWS - NR)
            glen = jnp.minimum(PAY, r1 - gstart)
            gstarts.append(gstart)
            aligneds.append(aligned)
            glens.append(glen)

        def start_copy(k):
            pltpu.async_copy(
                x_hbm.at[pl.ds(aligneds[k], NR)], bufs[k], sems[k]
            )

        def wait_copy(k):
            pltpu.make_async_copy(
                x_hbm.at[pl.ds(aligneds[k], NR)], bufs[k], sems[k]
            ).wait()

        for k in range(NBUF):
            @pl.when(glens[k] > 0)
            def _(k=k):
                start_copy(k)

        seg_end = [lane(ends, seg0 + i) for i in range(SEGC)]
        seg_len = [lane(lv, seg0 + i) for i in range(SEGC)]

        zf = jnp.zeros((L,), jnp.float32)
        for i in range(B):
            for j in range(NVEC):
                part[i, pl.ds(L * j, L)] = zf

        @pl.when(s == 0)
        def _():
            pltpu.sync_copy(part, shared)
        plsc.subcore_barrier()

        def process(k):
            buf = bufs[k]
            gstart, glen = gstarts[k], glens[k]
            off = gstart - aligneds[k]
            gend = gstart + glen
            for i in range(SEGC):
                r = seg0 + i
                lo = jnp.maximum(seg_end[i] - seg_len[i], gstart)
                hi = jnp.minimum(seg_end[i], gend)
                n = hi - lo

                @pl.when(n > 0)
                def _():
                    base = off + (lo - gstart)

                    def row_body(q, a):
                        p = base + q
                        return tuple(
                            a[j] + buf[p, pl.ds(L * j, L)]
                            for j in range(NVEC)
                        )

                    accs = lax.fori_loop(
                        0, n, row_body,
                        tuple(zf for _ in range(NVEC)),
                    )
                    for j in range(NVEC):
                        part[r, pl.ds(L * j, L)] = (
                            part[r, pl.ds(L * j, L)] + accs[j]
                        )

        for k in range(MAXC):
            @pl.when(glens[k] > 0)
            def _(k=k):
                wait_copy(k)
                process(k)
            if k + NBUF < MAXC:
                @pl.when(glens[k + NBUF] > 0)
                def _(k=k):
                    start_copy(k + NBUF)

        pltpu.sync_copy(part, shared.at[idx_vmem], add=True)
        plsc.subcore_barrier()

        @pl.when(s < SEGC)
        def _():
            r = seg0 + s
            pltpu.sync_copy(shared.at[r], row_vmem)
            den = jnp.full((L,), lane(lv, r), jnp.float32)
            for j in range(NVEC):
                row_vmem[pl.ds(L * j, L)] = row_vmem[pl.ds(L * j, L)] / den
            pltpu.sync_copy(row_vmem, out_hbm.at[r])

    return kern(x, batch_lengths)


def kernel(x, batch_lengths):
    lens = batch_lengths.astype(jnp.int32)
    return _sc_segment_means(x, lens)


# speedup vs baseline: 1.3638x; 1.1210x over previous
import dataclasses
import functools
import jax
import jax.numpy as jnp
from jax import lax
from jax.experimental import pallas as pl
from jax.experimental.pallas import tpu as pltpu
from jax.experimental.pallas import tpu_sc as plsc

N_ROWS = 32768
D = 128
B = 16
L = 16
NVEC = D // L
NWC = 16
SEGC = B // 2
NR = 256
PAY = NR - 8
NBUF = 3


def _sc_segment_means(x, batch_lengths):
    mesh = plsc.VectorSubcoreMesh(
        core_axis_name="c", subcore_axis_name="s", num_cores=2, num_subcores=16
    )
    cp = pltpu.CompilerParams()
    if "needs_layout_passes" in pltpu.CompilerParams.__dataclass_fields__:
        cp = dataclasses.replace(cp, needs_layout_passes=False)

    @functools.partial(
        pl.kernel,
        out_type=jax.ShapeDtypeStruct((B, D), jnp.float32),
        mesh=mesh,
        scratch_types=[
            pltpu.VMEM((B,), jnp.int32),
            pltpu.VMEM((B,), jnp.int32),
            pltpu.VMEM((NR, D), jnp.float32),
            pltpu.VMEM((NR, D), jnp.float32),
            pltpu.VMEM((NR, D), jnp.float32),
            pltpu.VMEM((B, D), jnp.float32),
            pltpu.VMEM((D,), jnp.float32),
            pltpu.VMEM_SHARED((B, D), jnp.float32),
            pltpu.SemaphoreType.DMA,
            pltpu.SemaphoreType.DMA,
            pltpu.SemaphoreType.DMA,
        ],
        compiler_params=cp,
    )
    def kern(x_hbm, len_hbm, out_hbm, len_vmem, idx_vmem,
             buf0, buf1, buf2, part, row_vmem, shared, sem0, sem1, sem2):
        c = lax.axis_index("c")
        s = lax.axis_index("s")

        pltpu.sync_copy(len_hbm, len_vmem)
        lv = len_vmem[...]
        ends = plsc.cumsum(lv)
        lanes = lax.iota(jnp.int32, L)
        idx_vmem[...] = lanes
        zeros_i = jnp.zeros((L,), jnp.int32)

        def lane(vec, i):
            return jnp.sum(jnp.where(lanes == i, vec, zeros_i))

        seg0 = c * SEGC
        core_lo = lane(ends, seg0 - 1)
        core_hi = lane(ends, seg0 + SEGC - 1)

        core_rows = core_hi - core_lo
        rows_per_w = (core_rows + NWC - 1) // NWC
        r0 = core_lo + s * rows_per_w
        r1 = jnp.minimum(r0 + rows_per_w, core_hi)
        cnt = jnp.maximum(r1 - r0, 0)
        nchunks = (cnt + PAY - 1) // PAY

        def chunk_geom(k):
            gstart = r0 + k * PAY
            aligned = jnp.minimum((gstart // 8) * 8, N_ROWS - NR)
            return gstart, aligned

        bufs = [buf0, buf1, buf2]
        sems = [sem0, sem1, sem2]

        def start_copy(k, slot):
            _, aligned = chunk_geom(k)
            pltpu.async_copy(
                x_hbm.at[pl.ds(aligned, NR)], bufs[slot], sems[slot]
            )

        for k in range(NBUF):
            @pl.when(k < nchunks)
            def _(k=k):
                start_copy(k, k)

        zf = jnp.zeros((L,), jnp.float32)
        for i in range(B):
            for j in range(NVEC):
                part[i, pl.ds(L * j, L)] = zf

        @pl.when(s == 0)
        def _():
            pltpu.sync_copy(part, shared)
        plsc.subcore_barrier()

        def make_chunk_fn(slot):
            buf = bufs[slot]
            sem = sems[slot]

            def run_chunk(k, t_in):
                gstart, aligned = chunk_geom(k)
                off = gstart - aligned
                gend = jnp.minimum(gstart + PAY, r1)
                pltpu.make_async_copy(
                    x_hbm.at[pl.ds(aligned, NR)], buf, sem
                ).wait()

                @pl.when(k + NBUF < nchunks)
                def _():
                    start_copy(k + NBUF, slot)

                def run_cond(st):
                    return st[0] < gend

                def run_body(st):
                    pos, t = st
                    t = lax.while_loop(
                        lambda tt: lane(ends, tt) <= pos,
                        lambda tt: tt + 1,
                        t,
                    )
                    run_end = jnp.minimum(lane(ends, t), gend)
                    n = run_end - pos
                    base = off + (pos - gstart)

                    def row_body(q, a):
                        p = base + q
                        return tuple(
                            a[j] + buf[p, pl.ds(L * j, L)]
                            for j in range(NVEC)
                        )
                    accs = lax.fori_loop(
                        0, n, row_body, tuple(zf for _ in range(NVEC))
                    )
                    for j in range(NVEC):
                        part[t, pl.ds(L * j, L)] = (
                            part[t, pl.ds(L * j, L)] + accs[j]
                        )
                    return (run_end, t)

                pos_t = lax.while_loop(run_cond, run_body, (gstart, t_in))
                return pos_t[1]
            return run_chunk

        chunk_fns = [make_chunk_fn(slot) for slot in range(NBUF)]

        def chunk_body(k, t):
            slot = k % NBUF
            return lax.cond(
                slot == 0,
                lambda: chunk_fns[0](k, t),
                lambda: lax.cond(
                    slot == 1,
                    lambda: chunk_fns[1](k, t),
                    lambda: chunk_fns[2](k, t),
                ),
            )

        lax.fori_loop(0, nchunks, chunk_body, seg0)

        pltpu.sync_copy(part, shared.at[idx_vmem], add=True)
        plsc.subcore_barrier()

        @pl.when(s < SEGC)
        def _():
            r = seg0 + s
            pltpu.sync_copy(shared.at[r], row_vmem)
            den = jnp.full((L,), lane(lv, r), jnp.float32)
            for j in range(NVEC):
                row_vmem[pl.ds(L * j, L)] = row_vmem[pl.ds(L * j, L)] / den
            pltpu.sync_copy(row_vmem, out_hbm.at[r])

    return kern(x, batch_lengths)


def kernel(x, batch_lengths):
    lens = batch_lengths.astype(jnp.int32)
    return _sc_segment_means(x, lens)
